# baseline (speedup 1.0000x reference)
import functools

import jax
import jax.numpy as jnp
from jax import lax
from jax.experimental import pallas as pl
from jax.experimental.pallas import tpu as pltpu
from jax.experimental.pallas import tpu_sc as plsc

_MAXLEN = 2048
_HEAD_DIM = 32
_TW = 2 * _MAXLEN * _HEAD_DIM
_ROW_W = _MAXLEN * _HEAD_DIM
_ROW_TILES = _ROW_W // 128

_info = plsc.get_sparse_core_info()
_NC = _info.num_cores
_NS = _info.num_subcores
_NW = _NC * _NS
_LANES = 16

_CLIP_W = _TW // _NW


@functools.partial(
    pl.kernel,
    mesh=plsc.VectorSubcoreMesh(core_axis_name="c", subcore_axis_name="s"),
    out_type=jax.ShapeDtypeStruct((_TW,), jnp.float32),
    scratch_types=[
        pltpu.VMEM((_CLIP_W,), jnp.float32),
    ],
)
def _clip_table(table_hbm, ctab_hbm, vbuf):
    cid = lax.axis_index("c")
    sid = lax.axis_index("s")
    wid = sid * _NC + cid
    base = wid * _CLIP_W
    pltpu.sync_copy(table_hbm.at[pl.ds(base, _CLIP_W)], vbuf)

    def clip_body(k, _):
        off = pl.multiple_of(k * _LANES, _LANES)
        v = vbuf[pl.ds(off, _LANES)]
        vbuf[pl.ds(off, _LANES)] = jnp.minimum(jnp.maximum(v, -5.0), 5.0)
        return 0

    lax.fori_loop(0, _CLIP_W // _LANES, clip_body, 0)
    pltpu.sync_copy(vbuf, ctab_hbm.at[pl.ds(base, _CLIP_W)])


_ROWS_PER_SCS = _MAXLEN // _NC


@functools.partial(
    pl.kernel,
    mesh=plsc.ScalarSubcoreMesh(axis_name="c", num_cores=2),
    out_type=jax.ShapeDtypeStruct((_MAXLEN * _ROW_TILES, 128), jnp.float32),
    scratch_types=[
        pltpu.VMEM_SHARED((4, _TW // 128, 128), jnp.float32),
        pltpu.SemaphoreType.DMA,
    ],
)
def _writeout_scs(phases_hbm, out_hbm, spmem, sem):
    cid = lax.axis_index("c")
    row0 = cid * _ROWS_PER_SCS
    pltpu.sync_copy(phases_hbm, spmem)

    _GRP = 8

    def _copy(m, r):
        i = row0 + (4 - m) % 4 + 4 * r
        q = (_MAXLEN - i - m) // 4
        return pltpu.make_async_copy(
            spmem.at[m, pl.ds(q, _ROW_TILES), :],
            out_hbm.at[pl.ds(i * _ROW_TILES, _ROW_TILES), :],
            sem,
        )

    for m in range(4):
        def group_body(g, _, m=m):
            for b in range(_GRP):
                _copy(m, g * _GRP + b).start()
            for b in range(_GRP):
                _copy(m, g * _GRP + b).wait()
            return 0

        lax.fori_loop(0, _ROWS_PER_SCS // 4 // _GRP, group_body, 0)


def kernel(pe_k_weight, seq_len):
    del seq_len
    flat = pe_k_weight.reshape(_TW)
    ctab = _clip_table(flat)
    padded = jnp.pad(ctab, (0, 96))
    phases = jnp.stack(
        [
            lax.dynamic_slice(padded, (32 * m,), (_TW,)).reshape(_TW // 128, 128)
            for m in range(4)
        ]
    )
    out = _writeout_scs(phases)
    return out.reshape(_MAXLEN, _MAXLEN, _HEAD_DIM)


# speedup vs baseline: 3.9076x; 3.9076x over previous
import functools

import jax
import jax.numpy as jnp
from jax import lax
from jax.experimental import pallas as pl
from jax.experimental.pallas import tpu as pltpu
from jax.experimental.pallas import tpu_sc as plsc

_MAXLEN = 2048
_HEAD_DIM = 32
_TROWS = 2 * _MAXLEN
_NPHASE = 128
_PADW = _TROWS + _NPHASE

_info = plsc.get_sparse_core_info()
_NC = _info.num_cores


def _phase_body(stab_ref, ph_ref):
    phi = pl.program_id(0)
    s = stab_ref[...]
    r = pltpu.roll(s, _PADW - phi, axis=1)
    ph_ref[0] = jnp.minimum(jnp.maximum(r[:, :_TROWS], -5.0), 5.0)


_phase_tc = pl.pallas_call(
    _phase_body,
    grid=(_NPHASE,),
    in_specs=[pl.BlockSpec((_HEAD_DIM, _PADW), lambda p: (0, 0))],
    out_specs=pl.BlockSpec((1, _HEAD_DIM, _TROWS), lambda p: (p, 0, 0)),
    out_shape=jax.ShapeDtypeStruct((_NPHASE, _HEAD_DIM, _TROWS), jnp.float32),
)

_GRP = 8


@functools.partial(
    pl.kernel,
    mesh=plsc.ScalarSubcoreMesh(axis_name="c", num_cores=_NC),
    out_type=jax.ShapeDtypeStruct((_MAXLEN, _HEAD_DIM, _MAXLEN), jnp.float32),
    scratch_types=[
        pltpu.VMEM_SHARED((_GRP, _HEAD_DIM, _TROWS), jnp.float32),
        pltpu.SemaphoreType.DMA,
    ],
)
def _writeout_scs(ph_hbm, out_hbm, spmem, sem):
    cid = lax.axis_index("c")

    def batch_body(b, _):
        pltpu.sync_copy(ph_hbm.at[pl.ds(b * _GRP, _GRP)], spmem)
        for g in range(_GRP):
            phi = b * _GRP + g
            kb = (1 - cid) * 8 + jnp.where(phi == 0, 1, 0)

            def _copy(t):
                k0 = kb + t
                i = _MAXLEN - phi - 128 * k0
                return pltpu.make_async_copy(
                    spmem.at[g, :, pl.ds(128 * k0, _MAXLEN)],
                    out_hbm.at[i],
                    sem,
                )

            for t in range(8):
                _copy(t).start()
            for t in range(8):
                _copy(t).wait()
        return 0

    lax.fori_loop(0, _NPHASE // _GRP, batch_body, 0)


def kernel(pe_k_weight, seq_len):
    del seq_len
    stab = jnp.pad(jnp.transpose(pe_k_weight), ((0, 0), (0, _NPHASE)))
    ph = _phase_tc(stab)
    p = _writeout_scs(ph)
    return jnp.transpose(p, (0, 2, 1))


# speedup vs baseline: 4.8824x; 1.2495x over previous
import functools

import jax
import jax.numpy as jnp
from jax import lax
from jax.experimental import pallas as pl
from jax.experimental.pallas import tpu as pltpu
from jax.experimental.pallas import tpu_sc as plsc

_MAXLEN = 2048
_HEAD_DIM = 32
_TROWS = 2 * _MAXLEN
_NPHASE = 128
_PADW = _TROWS + _NPHASE

_info = plsc.get_sparse_core_info()
_NC = _info.num_cores


_PPB = 4


def _phase_body(stab_ref, ph_ref):
    pid = pl.program_id(0)
    s = stab_ref[...]
    for u in range(_PPB):
        phi = pid * _PPB + u
        r = pltpu.roll(s, _PADW - phi, axis=1)
        ph_ref[u] = jnp.minimum(jnp.maximum(r[:, :_TROWS], -5.0), 5.0)


_phase_tc = pl.pallas_call(
    _phase_body,
    grid=(_NPHASE // _PPB,),
    in_specs=[pl.BlockSpec((_HEAD_DIM, _PADW), lambda p: (0, 0))],
    out_specs=pl.BlockSpec((_PPB, _HEAD_DIM, _TROWS), lambda p: (p, 0, 0)),
    out_shape=jax.ShapeDtypeStruct((_NPHASE, _HEAD_DIM, _TROWS), jnp.float32),
)

_GRP = 4
_NBAT = _NPHASE // _GRP


@functools.partial(
    pl.kernel,
    mesh=plsc.ScalarSubcoreMesh(axis_name="c", num_cores=_NC),
    out_type=jax.ShapeDtypeStruct((_MAXLEN, _HEAD_DIM, _MAXLEN), jnp.float32),
    scratch_types=[
        pltpu.VMEM_SHARED((2, _GRP, _HEAD_DIM, _TROWS), jnp.float32),
        pltpu.SemaphoreType.DMA,
        pltpu.SemaphoreType.DMA,
    ],
)
def _writeout_scs(ph_hbm, out_hbm, spmem, stage_sem, row_sem):
    cid = lax.axis_index("c")

    def _stage(b):
        return pltpu.make_async_copy(
            ph_hbm.at[pl.ds(b * _GRP, _GRP)],
            spmem.at[lax.rem(b, 2)],
            stage_sem,
        )

    _stage(0).start()

    def batch_body(b, _):
        _stage(b).wait()

        @pl.when(b + 1 < _NBAT)
        def _():
            _stage(b + 1).start()

        for g in range(_GRP):
            phi = b * _GRP + g
            kb = (1 - cid) * 8 + jnp.where(phi == 0, 1, 0)

            def _copy(t):
                k0 = kb + t
                i = _MAXLEN - phi - 128 * k0
                return pltpu.make_async_copy(
                    spmem.at[lax.rem(b, 2), g, :, pl.ds(128 * k0, _MAXLEN)],
                    out_hbm.at[i],
                    row_sem,
                )

            for t in range(8):
                _copy(t).start()
            for t in range(8):
                _copy(t).wait()
        return 0

    lax.fori_loop(0, _NBAT, batch_body, 0)


def kernel(pe_k_weight, seq_len):
    del seq_len
    stab = jnp.pad(jnp.transpose(pe_k_weight), ((0, 0), (0, _NPHASE)))
    ph = _phase_tc(stab)
    p = _writeout_scs(ph)
    return jnp.transpose(p, (0, 2, 1))


# speedup vs baseline: 6.1922x; 1.2683x over previous
import functools

import jax
import jax.numpy as jnp
from jax import lax
from jax.experimental import pallas as pl
from jax.experimental.pallas import tpu as pltpu
from jax.experimental.pallas import tpu_sc as plsc

_MAXLEN = 2048
_HEAD_DIM = 32
_TROWS = 2 * _MAXLEN
_NPHASE = 128
_PADW = _TROWS + _NPHASE

_info = plsc.get_sparse_core_info()
_NC = _info.num_cores


_PPB = 8


def _phase_body(stab_ref, ph_ref):
    pid = pl.program_id(0)
    s = stab_ref[...]
    for u in range(_PPB):
        phi = pid * _PPB + u
        r = pltpu.roll(s, _PADW - phi, axis=1)
        ph_ref[u] = jnp.minimum(jnp.maximum(r[:, :_TROWS], -5.0), 5.0)


_phase_tc = pl.pallas_call(
    _phase_body,
    grid=(_NPHASE // _PPB,),
    in_specs=[pl.BlockSpec((_HEAD_DIM, _PADW), lambda p: (0, 0))],
    out_specs=pl.BlockSpec((_PPB, _HEAD_DIM, _TROWS), lambda p: (p, 0, 0)),
    out_shape=jax.ShapeDtypeStruct((_NPHASE, _HEAD_DIM, _TROWS), jnp.float32),
)

_GRP = 4
_NBAT = _NPHASE // _GRP


@functools.partial(
    pl.kernel,
    mesh=plsc.ScalarSubcoreMesh(axis_name="c", num_cores=_NC),
    out_type=jax.ShapeDtypeStruct((_MAXLEN, _HEAD_DIM, _MAXLEN), jnp.float32),
    scratch_types=[
        pltpu.VMEM_SHARED((2, _GRP, _HEAD_DIM, _TROWS), jnp.float32),
        pltpu.SemaphoreType.DMA,
        pltpu.SemaphoreType.DMA,
    ],
)
def _writeout_scs(ph_hbm, out_hbm, spmem, stage_sem, row_sem):
    cid = lax.axis_index("c")

    def _stage(b):
        return pltpu.make_async_copy(
            ph_hbm.at[pl.ds(b * _GRP, _GRP)],
            spmem.at[lax.rem(b, 2)],
            stage_sem,
        )

    _stage(0).start()

    def batch_body(b, _):
        _stage(b).wait()

        @pl.when(b + 1 < _NBAT)
        def _():
            _stage(b + 1).start()

        def _copy(g, t):
            phi = b * _GRP + g
            kb = (1 - cid) * 8 + jnp.where(phi == 0, 1, 0)
            k0 = kb + t
            i = _MAXLEN - phi - 128 * k0
            return pltpu.make_async_copy(
                spmem.at[lax.rem(b, 2), g, :, pl.ds(128 * k0, _MAXLEN)],
                out_hbm.at[i],
                row_sem,
            )

        for g in range(_GRP):
            for t in range(8):
                _copy(g, t).start()
        for g in range(_GRP):
            for t in range(8):
                _copy(g, t).wait()
        return 0

    lax.fori_loop(0, _NBAT, batch_body, 0)


def kernel(pe_k_weight, seq_len):
    del seq_len
    stab = jnp.pad(jnp.transpose(pe_k_weight), ((0, 0), (0, _NPHASE)))
    ph = _phase_tc(stab)
    p = _writeout_scs(ph)
    return jnp.transpose(p, (0, 2, 1))


# speedup vs baseline: 7.0957x; 1.1459x over previous
import functools

import jax
import jax.numpy as jnp
from jax import lax
from jax.experimental import pallas as pl
from jax.experimental.pallas import tpu as pltpu
from jax.experimental.pallas import tpu_sc as plsc

_MAXLEN = 2048
_HEAD_DIM = 32
_TROWS = 2 * _MAXLEN
_NPHASE = 128
_PADW = _TROWS + _NPHASE

_info = plsc.get_sparse_core_info()
_NC = _info.num_cores


_PPB = 8


def _phase_body(stab_ref, ph_ref):
    pid = pl.program_id(0)
    s = stab_ref[...]
    for u in range(_PPB):
        phi = pid * _PPB + u
        r = pltpu.roll(s, _PADW - phi, axis=1)
        ph_ref[u] = jnp.minimum(jnp.maximum(r[:, :_TROWS], -5.0), 5.0)


_phase_tc = pl.pallas_call(
    _phase_body,
    grid=(_NPHASE // _PPB,),
    in_specs=[pl.BlockSpec((_HEAD_DIM, _PADW), lambda p: (0, 0))],
    out_specs=pl.BlockSpec((_PPB, _HEAD_DIM, _TROWS), lambda p: (p, 0, 0)),
    out_shape=jax.ShapeDtypeStruct((_NPHASE, _HEAD_DIM, _TROWS), jnp.float32),
)

_GRP = 2
_NBAT = _NPHASE // _GRP


@functools.partial(
    pl.kernel,
    mesh=plsc.ScalarSubcoreMesh(axis_name="c", num_cores=_NC),
    out_type=jax.ShapeDtypeStruct((_MAXLEN, _HEAD_DIM, _MAXLEN), jnp.float32),
    scratch_types=[
        pltpu.VMEM_SHARED((4, _GRP, _HEAD_DIM, _TROWS), jnp.float32),
        pltpu.SemaphoreType.DMA,
        pltpu.SemaphoreType.DMA,
    ],
)
def _writeout_scs(ph_hbm, out_hbm, spmem, stage_sem, row_sem):
    cid = lax.axis_index("c")

    def _stage(b):
        return pltpu.make_async_copy(
            ph_hbm.at[pl.ds(b * _GRP, _GRP)],
            spmem.at[lax.rem(b, 4)],
            stage_sem,
        )

    def _copy(b, g, t):
        phi = b * _GRP + g
        kb = (1 - cid) * 8 + jnp.where(phi == 0, 1, 0)
        k0 = kb + t
        i = _MAXLEN - phi - 128 * k0
        return pltpu.make_async_copy(
            spmem.at[lax.rem(b, 4), g, :, pl.ds(128 * k0, _MAXLEN)],
            out_hbm.at[i],
            row_sem,
        )

    def _fire_rows(b):
        for g in range(_GRP):
            for t in range(8):
                _copy(b, g, t).start()

    def _drain_rows(b):
        for g in range(_GRP):
            for t in range(8):
                _copy(b, g, t).wait()

    _stage(0).start()
    _stage(1).start()

    def batch_body(b, _):
        _stage(b).wait()

        @pl.when(b >= 2)
        def _():
            _drain_rows(b - 2)

        @pl.when(b + 2 < _NBAT)
        def _():
            _stage(b + 2).start()

        _fire_rows(b)
        return 0

    lax.fori_loop(0, _NBAT, batch_body, 0)
    _drain_rows(_NBAT - 2)
    _drain_rows(_NBAT - 1)


def kernel(pe_k_weight, seq_len):
    del seq_len
    stab = jnp.pad(jnp.transpose(pe_k_weight), ((0, 0), (0, _NPHASE)))
    ph = _phase_tc(stab)
    p = _writeout_scs(ph)
    return jnp.transpose(p, (0, 2, 1))


# speedup vs baseline: 7.1202x; 1.0035x over previous
import functools

import jax
import jax.numpy as jnp
from jax import lax
from jax.experimental import pallas as pl
from jax.experimental.pallas import tpu as pltpu
from jax.experimental.pallas import tpu_sc as plsc

_MAXLEN = 2048
_HEAD_DIM = 32
_TROWS = 2 * _MAXLEN
_NPHASE = 128
_PADW = _TROWS + _NPHASE
_HALF = _NPHASE // 2

_info = plsc.get_sparse_core_info()
_NC = _info.num_cores

_PPB = 8


def _make_phase_tc(base):
    def _phase_body(stab_ref, ph_ref):
        pid = pl.program_id(0)
        s = stab_ref[...]
        for u in range(_PPB):
            phi = base + pid * _PPB + u
            r = pltpu.roll(s, _PADW - phi, axis=1)
            ph_ref[u] = jnp.minimum(jnp.maximum(r[:, :_TROWS], -5.0), 5.0)

    return pl.pallas_call(
        _phase_body,
        grid=(_HALF // _PPB,),
        in_specs=[pl.BlockSpec((_HEAD_DIM, _PADW), lambda p: (0, 0))],
        out_specs=pl.BlockSpec((_PPB, _HEAD_DIM, _TROWS), lambda p: (p, 0, 0)),
        out_shape=jax.ShapeDtypeStruct((_HALF, _HEAD_DIM, _TROWS), jnp.float32),
    )


_GRP = 2
_NBAT = _HALF // _GRP


def _make_writeout(base):
    @functools.partial(
        pl.kernel,
        mesh=plsc.ScalarSubcoreMesh(axis_name="c", num_cores=_NC),
        scratch_types=[
            pltpu.VMEM_SHARED((4, _GRP, _HEAD_DIM, _TROWS), jnp.float32),
            pltpu.SemaphoreType.DMA,
            pltpu.SemaphoreType.DMA,
        ],
    )
    def _writeout_scs(ph_hbm, out_hbm, spmem, stage_sem, row_sem):
        cid = lax.axis_index("c")

        def _stage(b):
            return pltpu.make_async_copy(
                ph_hbm.at[pl.ds(b * _GRP, _GRP)],
                spmem.at[lax.rem(b, 4)],
                stage_sem,
            )

        def _copy(b, g, t):
            phi = base + b * _GRP + g
            kb = (1 - cid) * 8 + jnp.where(phi == 0, 1, 0)
            k0 = kb + t
            i = _MAXLEN - phi - 128 * k0
            return pltpu.make_async_copy(
                spmem.at[lax.rem(b, 4), g, :, pl.ds(128 * k0, _MAXLEN)],
                out_hbm.at[i],
                row_sem,
            )

        def _fire_rows(b):
            for g in range(_GRP):
                for t in range(8):
                    _copy(b, g, t).start()

        def _drain_rows(b):
            for g in range(_GRP):
                for t in range(8):
                    _copy(b, g, t).wait()

        _stage(0).start()
        _stage(1).start()

        def batch_body(b, _):
            _stage(b).wait()

            @pl.when(b >= 2)
            def _():
                _drain_rows(b - 2)

            @pl.when(b + 2 < _NBAT)
            def _():
                _stage(b + 2).start()

            _fire_rows(b)
            return 0

        lax.fori_loop(0, _NBAT, batch_body, 0)
        _drain_rows(_NBAT - 2)
        _drain_rows(_NBAT - 1)

    return _writeout_scs


_phase_tc_a = _make_phase_tc(0)
_phase_tc_b = _make_phase_tc(_HALF)
_writeout_a = _make_writeout(0)
_writeout_b = _make_writeout(_HALF)


def kernel(pe_k_weight, seq_len):
    del seq_len
    stab = jnp.pad(jnp.transpose(pe_k_weight), ((0, 0), (0, _NPHASE)))
    ph_a = _phase_tc_a(stab)
    out_ref = jax.new_ref(
        lax.empty((_MAXLEN, _HEAD_DIM, _MAXLEN), jnp.float32)
    )
    _writeout_a(ph_a, out_ref)
    ph_b = _phase_tc_b(stab)
    _writeout_b(ph_b, out_ref)
    return jnp.transpose(out_ref[...], (0, 2, 1))
